# baseline (speedup 1.0000x reference)
import functools
import math

import jax
import jax.numpy as jnp
from jax import lax
from jax.experimental import pallas as pl
from jax.experimental.pallas import tpu as pltpu
from jax.experimental.pallas import tpu_sc as plsc

N = 10000
E = 320000
D = 128
H = 8
C = 64
HD = C // H
RH = 32

BN = 1000
BE = 2000

_INTERPRET = False


def _head_matrix():
    r = lax.broadcasted_iota(jnp.int32, (C, H), 0)
    c = lax.broadcasted_iota(jnp.int32, (C, H), 1)
    return (r // HD == c).astype(jnp.float32)


def _node_pre_body(x_ref, wq_ref, wk_ref, wv_ref, q_ref, kv_ref):
    x = x_ref[...]
    q = jnp.dot(x, wq_ref[...], preferred_element_type=jnp.float32)
    q_ref[...] = q.astype(jnp.bfloat16)
    k = jnp.dot(x, wk_ref[...], preferred_element_type=jnp.float32)
    v = jnp.dot(x, wv_ref[...], preferred_element_type=jnp.float32)
    kv_ref[...] = jnp.concatenate([k, v], axis=1).astype(jnp.bfloat16)


def _node_pre(x, wq, wk, wv):
    return pl.pallas_call(
        _node_pre_body,
        grid=(N // BN,),
        in_specs=[
            pl.BlockSpec((BN, D), lambda i: (i, 0)),
            pl.BlockSpec((D, C), lambda i: (0, 0)),
            pl.BlockSpec((D, C), lambda i: (0, 0)),
            pl.BlockSpec((D, C), lambda i: (0, 0)),
        ],
        out_specs=[
            pl.BlockSpec((BN, C), lambda i: (i, 0)),
            pl.BlockSpec((BN, 2 * C), lambda i: (i, 0)),
        ],
        out_shape=[
            jax.ShapeDtypeStruct((N, C), jnp.bfloat16),
            jax.ShapeDtypeStruct((N, 2 * C), jnp.bfloat16),
        ],
        compiler_params=pltpu.CompilerParams(
            dimension_semantics=("parallel",)),
        interpret=_INTERPRET,
    )(x, wq, wk, wv)


def _edge_attn_body(geq_ref, gekv_ref, rp_ref, sc_ref,
                    rk1_ref, rk2_ref, rv1_ref, rv2_ref, out_ref):
    rp = rp_ref[...]
    dist = jnp.sqrt(jnp.sum(rp * rp, axis=1, keepdims=True))
    hk = jax.nn.relu(dist * rk1_ref[...])
    rk = jnp.dot(hk, rk2_ref[...], preferred_element_type=jnp.float32)
    hv = jax.nn.relu(dist * rv1_ref[...])
    rv = jnp.dot(hv, rv2_ref[...], preferred_element_type=jnp.float32)
    kv = gekv_ref[...].astype(jnp.float32)
    k = kv[:, :C] * rk
    v = kv[:, C:] * rv
    prod = geq_ref[...].astype(jnp.float32) * k
    S = _head_matrix()
    logits = jnp.dot(prod, S, preferred_element_type=jnp.float32) / math.sqrt(HD)
    ex = jnp.exp(logits) * sc_ref[...]
    ex64 = jnp.dot(ex, S.T, preferred_element_type=jnp.float32)
    wmsg = ex64 * v
    zeros = jnp.zeros((wmsg.shape[0], 8), jnp.float32)
    out_ref[...] = jnp.concatenate([wmsg, ex, zeros], axis=1)


def _edge_attn(geq, gekv, rel_pos, scale2d, rk1, rk2, rv1, rv2):
    return pl.pallas_call(
        _edge_attn_body,
        grid=(E // BE,),
        in_specs=[
            pl.BlockSpec((BE, C), lambda i: (i, 0)),
            pl.BlockSpec((BE, 2 * C), lambda i: (i, 0)),
            pl.BlockSpec((BE, 3), lambda i: (i, 0)),
            pl.BlockSpec((BE, 1), lambda i: (i, 0)),
            pl.BlockSpec((1, RH), lambda i: (0, 0)),
            pl.BlockSpec((RH, C), lambda i: (0, 0)),
            pl.BlockSpec((1, RH), lambda i: (0, 0)),
            pl.BlockSpec((RH, C), lambda i: (0, 0)),
        ],
        out_specs=pl.BlockSpec((BE, 80), lambda i: (i, 0)),
        out_shape=jax.ShapeDtypeStruct((E, 80), jnp.float32),
        compiler_params=pltpu.CompilerParams(
            dimension_semantics=("parallel",)),
        interpret=_INTERPRET,
    )(geq, gekv, rel_pos, scale2d, rk1, rk2, rv1, rv2)


def _node_post_body(p_ref, x_ref, wo_ref, wf_ref, xn_ref, xf_ref):
    agg = p_ref[0] + p_ref[1]
    S = _head_matrix()
    den64 = jnp.dot(agg[:, C:C + H], S.T, preferred_element_type=jnp.float32)
    attn_agg = agg[:, :C] / (den64 + 1e-9)
    xn = x_ref[...] + jnp.dot(attn_agg, wo_ref[...],
                              preferred_element_type=jnp.float32)
    xn_ref[...] = xn
    xf = jnp.dot(xn, wf_ref[...], preferred_element_type=jnp.float32)
    xf_ref[...] = xf.astype(jnp.bfloat16)


def _node_post(parts, x, wo, wf):
    return pl.pallas_call(
        _node_post_body,
        grid=(N // BN,),
        in_specs=[
            pl.BlockSpec((2, BN, 80), lambda i: (0, i, 0)),
            pl.BlockSpec((BN, D), lambda i: (i, 0)),
            pl.BlockSpec((C, D), lambda i: (0, 0)),
            pl.BlockSpec((D, D), lambda i: (0, 0)),
        ],
        out_specs=[
            pl.BlockSpec((BN, D), lambda i: (i, 0)),
            pl.BlockSpec((BN, D), lambda i: (i, 0)),
        ],
        out_shape=[
            jax.ShapeDtypeStruct((N, D), jnp.float32),
            jax.ShapeDtypeStruct((N, D), jnp.bfloat16),
        ],
        compiler_params=pltpu.CompilerParams(
            dimension_semantics=("parallel",)),
        interpret=_INTERPRET,
    )(parts, x, wo, wf)


def _edge_final_body(gef_ref, rp_ref, sc_ref, rf1_ref, rf2_ref,
                     outa_ref, outb_ref):
    rp = rp_ref[...]
    dist = jnp.sqrt(jnp.sum(rp * rp, axis=1, keepdims=True))
    hf = jax.nn.relu(dist * rf1_ref[...])
    rf = jnp.dot(hf, rf2_ref[...], preferred_element_type=jnp.float32)
    msg = gef_ref[...].astype(jnp.float32) * rf * sc_ref[...]
    outa_ref[...] = msg[:, :C]
    outb_ref[...] = msg[:, C:]


def _edge_final(gef, rel_pos, scale2d, rf1, rf2):
    return pl.pallas_call(
        _edge_final_body,
        grid=(E // BE,),
        in_specs=[
            pl.BlockSpec((BE, D), lambda i: (i, 0)),
            pl.BlockSpec((BE, 3), lambda i: (i, 0)),
            pl.BlockSpec((BE, 1), lambda i: (i, 0)),
            pl.BlockSpec((1, RH), lambda i: (0, 0)),
            pl.BlockSpec((RH, D), lambda i: (0, 0)),
        ],
        out_specs=[
            pl.BlockSpec((BE, C), lambda i: (i, 0)),
            pl.BlockSpec((BE, C), lambda i: (i, 0)),
        ],
        out_shape=[
            jax.ShapeDtypeStruct((E, C), jnp.float32),
            jax.ShapeDtypeStruct((E, C), jnp.float32),
        ],
        compiler_params=pltpu.CompilerParams(
            dimension_semantics=("parallel",)),
        interpret=_INTERPRET,
    )(gef, rel_pos, scale2d, rf1, rf2)


def _final_sum_body(pa_ref, pb_ref, out_ref):
    a = pa_ref[0] + pa_ref[1]
    b = pb_ref[0] + pb_ref[1]
    out_ref[...] = jnp.concatenate([a, b], axis=1)


def _final_sum(pa, pb):
    return pl.pallas_call(
        _final_sum_body,
        grid=(N // BN,),
        in_specs=[
            pl.BlockSpec((2, BN, C), lambda i: (0, i, 0)),
            pl.BlockSpec((2, BN, C), lambda i: (0, i, 0)),
        ],
        out_specs=pl.BlockSpec((BN, D), lambda i: (i, 0)),
        out_shape=jax.ShapeDtypeStruct((N, D), jnp.float32),
        compiler_params=pltpu.CompilerParams(
            dimension_semantics=("parallel",)),
        interpret=_INTERPRET,
    )(pa, pb)


_NC = 2
_NS = 16
_NW = _NC * _NS
_EPW = E // _NW
_CH = 80
_NCHK = _EPW // _CH
_NBUF = 5
_NGRP = _NCHK // _NBUF
_NPT = N // _NS


def _sc_mesh():
    return plsc.VectorSubcoreMesh(core_axis_name="c", subcore_axis_name="s")


def _sc_gather(streams):
    widths = [int(t.shape[1]) for t, _ in streams]
    dtypes = [t.dtype for t, _ in streams]
    ns = len(streams)

    @functools.partial(
        pl.kernel,
        out_type=[jax.ShapeDtypeStruct((E, w), dt)
                  for w, dt in zip(widths, dtypes)],
        mesh=_sc_mesh(),
        scratch_types=[pltpu.VMEM((_NCHK, _CH), jnp.int32) for _ in range(ns)] +
                      [pltpu.VMEM((_NBUF, _CH, w), dt)
                       for w, dt in zip(widths, dtypes)] +
                      [pltpu.SemaphoreType.DMA] * (2 * _NBUF),
        compiler_params=pltpu.CompilerParams(use_tc_tiling_on_sc=False),
    )
    def k(*refs):
        tabs = refs[0:2 * ns:2]
        idxs = refs[1:2 * ns:2]
        outs = refs[2 * ns:3 * ns]
        idx_all = refs[3 * ns:4 * ns]
        rows = refs[4 * ns:5 * ns]
        gsems = refs[5 * ns:5 * ns + _NBUF]
        wsems = refs[5 * ns + _NBUF:5 * ns + 2 * _NBUF]
        c = lax.axis_index("c")
        s = lax.axis_index("s")
        wid = s * _NC + c
        base = wid * _EPW

        def one_stream(i_hbm, t_hbm, o_hbm, i_v, r_v):
            pltpu.sync_copy(i_hbm.at[pl.ds(wid * _NCHK, _NCHK)], i_v)

            def fire(j, p):
                pltpu.async_copy(t_hbm.at[i_v.at[j]], r_v.at[p], gsems[p])

            def drain_write(j, p):
                pltpu.make_async_copy(t_hbm.at[i_v.at[0]], r_v.at[p],
                                      gsems[p]).wait()
                pltpu.async_copy(r_v.at[p],
                                 o_hbm.at[pl.ds(base + j * _CH, _CH)],
                                 wsems[p])

            def wait_write(p):
                pltpu.make_async_copy(r_v.at[p],
                                      o_hbm.at[pl.ds(base, _CH)],
                                      wsems[p]).wait()

            @pl.loop(0, _NGRP)
            def _(g):
                for p in range(_NBUF):
                    j = g * _NBUF + p

                    @pl.when(g > 0)
                    def _():
                        wait_write(p)
                    fire(j, p)
                    q = (p - 2) % _NBUF
                    if p >= 2:
                        drain_write(j - 2, q)
                    else:
                        @pl.when(g > 0)
                        def _():
                            drain_write(j - 2, q)

            drain_write(_NCHK - 2, (_NCHK - 2) % _NBUF)
            drain_write(_NCHK - 1, (_NCHK - 1) % _NBUF)
            for p in range(_NBUF):
                wait_write(p)

        for j in range(ns):
            one_stream(idxs[j], tabs[j], outs[j], idx_all[j], rows[j])

    flat = []
    for t, i in streams:
        flat += [t, i]
    return k(*flat)


def _sc_scatter_add(rows, idx, w, zeros):
    @functools.partial(
        pl.kernel,
        out_type=jax.ShapeDtypeStruct((_NC, N, w), jnp.float32),
        mesh=_sc_mesh(),
        scratch_types=[pltpu.VMEM((_NCHK, _CH), jnp.int32),
                       pltpu.VMEM((_NBUF, _CH, w), jnp.float32),
                       pltpu.VMEM_SHARED((N, w), jnp.float32)] +
                      [pltpu.SemaphoreType.DMA] * (2 * _NBUF),
        compiler_params=pltpu.CompilerParams(use_tc_tiling_on_sc=False),
    )
    def k(r_hbm, i_hbm, z_hbm, o_hbm, i_v, r_v, acc_sh, *sems):
        rsems = sems[0:_NBUF]
        ssems = sems[_NBUF:2 * _NBUF]
        c = lax.axis_index("c")
        s = lax.axis_index("s")
        wid = s * _NC + c
        base = wid * _EPW
        pltpu.sync_copy(i_hbm.at[pl.ds(wid * _NCHK, _NCHK)], i_v)
        pltpu.sync_copy(z_hbm.at[pl.ds(s * _NPT, _NPT)],
                        acc_sh.at[pl.ds(s * _NPT, _NPT)])
        plsc.subcore_barrier()

        def load_rows(j, p):
            pltpu.async_copy(r_hbm.at[pl.ds(base + j * _CH, _CH)],
                             r_v.at[p], rsems[p])

        def wait_rows(p):
            pltpu.make_async_copy(r_hbm.at[pl.ds(base, _CH)],
                                  r_v.at[p], rsems[p]).wait()

        def fire_scatter(j, p):
            pltpu.async_copy(r_v.at[p], acc_sh.at[i_v.at[j]],
                             ssems[p], add=True)

        def wait_scatter(p):
            pltpu.make_async_copy(r_v.at[p], acc_sh.at[i_v.at[0]],
                                  ssems[p]).wait()

        load_rows(0, 0)
        load_rows(1, 1)

        @pl.loop(0, _NGRP)
        def _(g):
            for p in range(_NBUF):
                j = g * _NBUF + p
                wait_rows(p)
                fire_scatter(j, p)
                q = (p + 2) % _NBUF
                if p < 3:
                    @pl.when(g > 0)
                    def _():
                        wait_scatter(q)
                    load_rows(j + 2, q)
                else:
                    wait_scatter(q)

                    @pl.when(j + 2 < _NCHK)
                    def _():
                        load_rows(j + 2, q)

        for p in range(2, _NBUF):
            wait_scatter(p)
        plsc.subcore_barrier()
        pltpu.sync_copy(acc_sh.at[pl.ds(s * _NPT, _NPT)],
                        o_hbm.at[c, pl.ds(s * _NPT, _NPT)])

    return k(rows, idx, zeros)


def kernel(node_feats, edge_index, rel_pos, scale, Wq, Wk, Wv, Wo,
           Rk1, Rk2, Rv1, Rv2, Wf, Rf1, Rf2):
    src = edge_index[0].astype(jnp.int32).reshape(E // _CH, _CH)
    dst = edge_index[1].astype(jnp.int32).reshape(E // _CH, _CH)
    scale2d = scale.reshape(E, 1)
    zeros80 = jnp.zeros((N, 80), jnp.float32)
    zeros64 = jnp.zeros((N, C), jnp.float32)

    x = node_feats
    for l in range(2):
        q, kv = _node_pre(x, Wq[l], Wk[l], Wv[l])
        geq, gekv = _sc_gather([(q, dst), (kv, src)])
        packed = _edge_attn(geq, gekv, rel_pos, scale2d,
                            Rk1[l], Rk2[l], Rv1[l], Rv2[l])
        parts = _sc_scatter_add(packed, dst, 80, zeros80)
        if l == 0:
            x, _ = _node_post(parts, x, Wo[l], Wf)
        else:
            x, xf = _node_post(parts, x, Wo[l], Wf)

    gef, = _sc_gather([(xf, src)])
    msga, msgb = _edge_final(gef, rel_pos, scale2d, Rf1, Rf2)
    fpa = _sc_scatter_add(msga, dst, C, zeros64)
    fpb = _sc_scatter_add(msgb, dst, C, zeros64)
    return _final_sum(fpa, fpb)


# speedup vs baseline: 1.3482x; 1.3482x over previous
import functools
import math

import jax
import jax.numpy as jnp
from jax import lax
from jax.experimental import pallas as pl
from jax.experimental.pallas import tpu as pltpu
from jax.experimental.pallas import tpu_sc as plsc

N = 10000
E = 320000
D = 128
H = 8
C = 64
HD = C // H
RH = 32

BN = 1000
BE = 2000

_INTERPRET = False


def _head_matrix():
    r = lax.broadcasted_iota(jnp.int32, (C, H), 0)
    c = lax.broadcasted_iota(jnp.int32, (C, H), 1)
    return (r // HD == c).astype(jnp.float32)


def _node_pre_body(x_ref, wq_ref, wk_ref, wv_ref, q_ref, kv_ref):
    x = x_ref[...]
    q_ref[...] = jnp.dot(x, wq_ref[...], preferred_element_type=jnp.float32)
    k = jnp.dot(x, wk_ref[...], preferred_element_type=jnp.float32)
    v = jnp.dot(x, wv_ref[...], preferred_element_type=jnp.float32)
    kv_ref[...] = jnp.concatenate([k, v], axis=1)


def _node_pre(x, wq, wk, wv):
    return pl.pallas_call(
        _node_pre_body,
        grid=(N // BN,),
        in_specs=[
            pl.BlockSpec((BN, D), lambda i: (i, 0)),
            pl.BlockSpec((D, C), lambda i: (0, 0)),
            pl.BlockSpec((D, C), lambda i: (0, 0)),
            pl.BlockSpec((D, C), lambda i: (0, 0)),
        ],
        out_specs=[
            pl.BlockSpec((BN, C), lambda i: (i, 0)),
            pl.BlockSpec((BN, 2 * C), lambda i: (i, 0)),
        ],
        out_shape=[
            jax.ShapeDtypeStruct((N, C), jnp.float32),
            jax.ShapeDtypeStruct((N, 2 * C), jnp.float32),
        ],
        compiler_params=pltpu.CompilerParams(
            dimension_semantics=("parallel",)),
        interpret=_INTERPRET,
    )(x, wq, wk, wv)


def _edge_attn_body(geq_ref, gekv_ref, rp_ref, sc_ref,
                    rk1_ref, rk2_ref, rv1_ref, rv2_ref, out_ref):
    rp = rp_ref[...]
    dist = jnp.sqrt(jnp.sum(rp * rp, axis=1, keepdims=True))
    hk = jax.nn.relu(dist * rk1_ref[...])
    rk = jnp.dot(hk, rk2_ref[...], preferred_element_type=jnp.float32)
    hv = jax.nn.relu(dist * rv1_ref[...])
    rv = jnp.dot(hv, rv2_ref[...], preferred_element_type=jnp.float32)
    kv = gekv_ref[...]
    k = kv[:, :C] * rk
    v = kv[:, C:] * rv
    prod = geq_ref[...] * k
    S = _head_matrix()
    logits = jnp.dot(prod, S, preferred_element_type=jnp.float32) / math.sqrt(HD)
    ex = jnp.exp(logits) * sc_ref[...]
    ex64 = jnp.dot(ex, S.T, preferred_element_type=jnp.float32)
    wmsg = ex64 * v
    zeros = jnp.zeros((wmsg.shape[0], 8), jnp.float32)
    out_ref[...] = jnp.concatenate([wmsg, ex, zeros], axis=1)


def _edge_attn(geq, gekv, rel_pos, scale2d, rk1, rk2, rv1, rv2):
    return pl.pallas_call(
        _edge_attn_body,
        grid=(E // BE,),
        in_specs=[
            pl.BlockSpec((BE, C), lambda i: (i, 0)),
            pl.BlockSpec((BE, 2 * C), lambda i: (i, 0)),
            pl.BlockSpec((BE, 3), lambda i: (i, 0)),
            pl.BlockSpec((BE, 1), lambda i: (i, 0)),
            pl.BlockSpec((1, RH), lambda i: (0, 0)),
            pl.BlockSpec((RH, C), lambda i: (0, 0)),
            pl.BlockSpec((1, RH), lambda i: (0, 0)),
            pl.BlockSpec((RH, C), lambda i: (0, 0)),
        ],
        out_specs=pl.BlockSpec((BE, 80), lambda i: (i, 0)),
        out_shape=jax.ShapeDtypeStruct((E, 80), jnp.float32),
        compiler_params=pltpu.CompilerParams(
            dimension_semantics=("parallel",)),
        interpret=_INTERPRET,
    )(geq, gekv, rel_pos, scale2d, rk1, rk2, rv1, rv2)


def _node_post_body(p_ref, x_ref, wo_ref, wf_ref, xn_ref, xf_ref):
    agg = p_ref[0] + p_ref[1]
    S = _head_matrix()
    den64 = jnp.dot(agg[:, C:C + H], S.T, preferred_element_type=jnp.float32)
    attn_agg = agg[:, :C] / (den64 + 1e-9)
    xn = x_ref[...] + jnp.dot(attn_agg, wo_ref[...],
                              preferred_element_type=jnp.float32)
    xn_ref[...] = xn
    xf_ref[...] = jnp.dot(xn, wf_ref[...], preferred_element_type=jnp.float32)


def _node_post(parts, x, wo, wf):
    return pl.pallas_call(
        _node_post_body,
        grid=(N // BN,),
        in_specs=[
            pl.BlockSpec((2, BN, 80), lambda i: (0, i, 0)),
            pl.BlockSpec((BN, D), lambda i: (i, 0)),
            pl.BlockSpec((C, D), lambda i: (0, 0)),
            pl.BlockSpec((D, D), lambda i: (0, 0)),
        ],
        out_specs=[
            pl.BlockSpec((BN, D), lambda i: (i, 0)),
            pl.BlockSpec((BN, D), lambda i: (i, 0)),
        ],
        out_shape=[
            jax.ShapeDtypeStruct((N, D), jnp.float32),
            jax.ShapeDtypeStruct((N, D), jnp.float32),
        ],
        compiler_params=pltpu.CompilerParams(
            dimension_semantics=("parallel",)),
        interpret=_INTERPRET,
    )(parts, x, wo, wf)


def _edge_final_body(gef_ref, rp_ref, sc_ref, rf1_ref, rf2_ref,
                     outa_ref, outb_ref):
    rp = rp_ref[...]
    dist = jnp.sqrt(jnp.sum(rp * rp, axis=1, keepdims=True))
    hf = jax.nn.relu(dist * rf1_ref[...])
    rf = jnp.dot(hf, rf2_ref[...], preferred_element_type=jnp.float32)
    msg = gef_ref[...] * rf * sc_ref[...]
    outa_ref[...] = msg[:, :C]
    outb_ref[...] = msg[:, C:]


def _edge_final(gef, rel_pos, scale2d, rf1, rf2):
    return pl.pallas_call(
        _edge_final_body,
        grid=(E // BE,),
        in_specs=[
            pl.BlockSpec((BE, D), lambda i: (i, 0)),
            pl.BlockSpec((BE, 3), lambda i: (i, 0)),
            pl.BlockSpec((BE, 1), lambda i: (i, 0)),
            pl.BlockSpec((1, RH), lambda i: (0, 0)),
            pl.BlockSpec((RH, D), lambda i: (0, 0)),
        ],
        out_specs=[
            pl.BlockSpec((BE, C), lambda i: (i, 0)),
            pl.BlockSpec((BE, C), lambda i: (i, 0)),
        ],
        out_shape=[
            jax.ShapeDtypeStruct((E, C), jnp.float32),
            jax.ShapeDtypeStruct((E, C), jnp.float32),
        ],
        compiler_params=pltpu.CompilerParams(
            dimension_semantics=("parallel",)),
        interpret=_INTERPRET,
    )(gef, rel_pos, scale2d, rf1, rf2)


def _final_sum_body(pa_ref, pb_ref, out_ref):
    a = pa_ref[0] + pa_ref[1]
    b = pb_ref[0] + pb_ref[1]
    out_ref[...] = jnp.concatenate([a, b], axis=1)


def _final_sum(pa, pb):
    return pl.pallas_call(
        _final_sum_body,
        grid=(N // BN,),
        in_specs=[
            pl.BlockSpec((2, BN, C), lambda i: (0, i, 0)),
            pl.BlockSpec((2, BN, C), lambda i: (0, i, 0)),
        ],
        out_specs=pl.BlockSpec((BN, D), lambda i: (i, 0)),
        out_shape=jax.ShapeDtypeStruct((N, D), jnp.float32),
        compiler_params=pltpu.CompilerParams(
            dimension_semantics=("parallel",)),
        interpret=_INTERPRET,
    )(pa, pb)


_NC = 2
_NS = 16
_NW = _NC * _NS
_EPW = E // _NW
_CH = 80
_NCHK = _EPW // _CH
_NBUF = 5
_NGRP = _NCHK // _NBUF
_NPT = N // _NS


def _sc_mesh():
    return plsc.VectorSubcoreMesh(core_axis_name="c", subcore_axis_name="s")


def _sc_gather(streams):
    widths = [int(t.shape[1]) for t, _ in streams]
    dtypes = [t.dtype for t, _ in streams]
    ns = len(streams)

    @functools.partial(
        pl.kernel,
        out_type=[jax.ShapeDtypeStruct((E, w), dt)
                  for w, dt in zip(widths, dtypes)],
        mesh=_sc_mesh(),
        scratch_types=[pltpu.VMEM((_NCHK, _CH), jnp.int32) for _ in range(ns)] +
                      [pltpu.VMEM((_NBUF, _CH, w), dt)
                       for w, dt in zip(widths, dtypes)] +
                      [pltpu.SemaphoreType.DMA] * (2 * _NBUF),
        compiler_params=pltpu.CompilerParams(use_tc_tiling_on_sc=False),
    )
    def k(*refs):
        tabs = refs[0:2 * ns:2]
        idxs = refs[1:2 * ns:2]
        outs = refs[2 * ns:3 * ns]
        idx_all = refs[3 * ns:4 * ns]
        rows = refs[4 * ns:5 * ns]
        gsems = refs[5 * ns:5 * ns + _NBUF]
        wsems = refs[5 * ns + _NBUF:5 * ns + 2 * _NBUF]
        c = lax.axis_index("c")
        s = lax.axis_index("s")
        wid = s * _NC + c
        base = wid * _EPW

        def one_stream(i_hbm, t_hbm, o_hbm, i_v, r_v):
            pltpu.sync_copy(i_hbm.at[pl.ds(wid * _NCHK, _NCHK)], i_v)

            def fire(j, p):
                pltpu.async_copy(t_hbm.at[i_v.at[j]], r_v.at[p], gsems[p])

            def drain_write(j, p):
                pltpu.make_async_copy(t_hbm.at[i_v.at[0]], r_v.at[p],
                                      gsems[p]).wait()
                pltpu.async_copy(r_v.at[p],
                                 o_hbm.at[pl.ds(base + j * _CH, _CH)],
                                 wsems[p])

            def wait_write(p):
                pltpu.make_async_copy(r_v.at[p],
                                      o_hbm.at[pl.ds(base, _CH)],
                                      wsems[p]).wait()

            @pl.loop(0, _NGRP)
            def _(g):
                for p in range(_NBUF):
                    j = g * _NBUF + p

                    @pl.when(g > 0)
                    def _():
                        wait_write(p)
                    fire(j, p)
                    q = (p - 2) % _NBUF
                    if p >= 2:
                        drain_write(j - 2, q)
                    else:
                        @pl.when(g > 0)
                        def _():
                            drain_write(j - 2, q)

            drain_write(_NCHK - 2, (_NCHK - 2) % _NBUF)
            drain_write(_NCHK - 1, (_NCHK - 1) % _NBUF)
            for p in range(_NBUF):
                wait_write(p)

        for j in range(ns):
            one_stream(idxs[j], tabs[j], outs[j], idx_all[j], rows[j])

    flat = []
    for t, i in streams:
        flat += [t, i]
    return k(*flat)


def _sc_scatter_add(rows, idx, w, zeros):
    @functools.partial(
        pl.kernel,
        out_type=jax.ShapeDtypeStruct((_NC, N, w), jnp.float32),
        mesh=_sc_mesh(),
        scratch_types=[pltpu.VMEM((_NCHK, _CH), jnp.int32),
                       pltpu.VMEM((_NBUF, _CH, w), jnp.float32),
                       pltpu.VMEM_SHARED((N, w), jnp.float32)] +
                      [pltpu.SemaphoreType.DMA] * (2 * _NBUF),
        compiler_params=pltpu.CompilerParams(use_tc_tiling_on_sc=False),
    )
    def k(r_hbm, i_hbm, z_hbm, o_hbm, i_v, r_v, acc_sh, *sems):
        rsems = sems[0:_NBUF]
        ssems = sems[_NBUF:2 * _NBUF]
        c = lax.axis_index("c")
        s = lax.axis_index("s")
        wid = s * _NC + c
        base = wid * _EPW
        pltpu.sync_copy(i_hbm.at[pl.ds(wid * _NCHK, _NCHK)], i_v)
        pltpu.sync_copy(z_hbm.at[pl.ds(s * _NPT, _NPT)],
                        acc_sh.at[pl.ds(s * _NPT, _NPT)])
        plsc.subcore_barrier()

        def load_rows(j, p):
            pltpu.async_copy(r_hbm.at[pl.ds(base + j * _CH, _CH)],
                             r_v.at[p], rsems[p])

        def wait_rows(p):
            pltpu.make_async_copy(r_hbm.at[pl.ds(base, _CH)],
                                  r_v.at[p], rsems[p]).wait()

        def fire_scatter(j, p):
            pltpu.async_copy(r_v.at[p], acc_sh.at[i_v.at[j]],
                             ssems[p], add=True)

        def wait_scatter(p):
            pltpu.make_async_copy(r_v.at[p], acc_sh.at[i_v.at[0]],
                                  ssems[p]).wait()

        load_rows(0, 0)
        load_rows(1, 1)

        @pl.loop(0, _NGRP)
        def _(g):
            for p in range(_NBUF):
                j = g * _NBUF + p
                wait_rows(p)
                fire_scatter(j, p)
                q = (p + 2) % _NBUF
                if p < 3:
                    @pl.when(g > 0)
                    def _():
                        wait_scatter(q)
                    load_rows(j + 2, q)
                else:
                    wait_scatter(q)

                    @pl.when(j + 2 < _NCHK)
                    def _():
                        load_rows(j + 2, q)

        for p in range(2, _NBUF):
            wait_scatter(p)
        plsc.subcore_barrier()
        pltpu.sync_copy(acc_sh.at[pl.ds(s * _NPT, _NPT)],
                        o_hbm.at[c, pl.ds(s * _NPT, _NPT)])

    return k(rows, idx, zeros)


def kernel(node_feats, edge_index, rel_pos, scale, Wq, Wk, Wv, Wo,
           Rk1, Rk2, Rv1, Rv2, Wf, Rf1, Rf2):
    src = edge_index[0].astype(jnp.int32).reshape(E // _CH, _CH)
    dst = edge_index[1].astype(jnp.int32).reshape(E // _CH, _CH)
    scale2d = scale.reshape(E, 1)
    zeros80 = jnp.zeros((N, 80), jnp.float32)
    zeros64 = jnp.zeros((N, C), jnp.float32)

    x = node_feats
    for l in range(2):
        q, kv = _node_pre(x, Wq[l], Wk[l], Wv[l])
        geq, gekv = _sc_gather([(q, dst), (kv, src)])
        packed = _edge_attn(geq, gekv, rel_pos, scale2d,
                            Rk1[l], Rk2[l], Rv1[l], Rv2[l])
        parts = _sc_scatter_add(packed, dst, 80, zeros80)
        if l == 0:
            x, _ = _node_post(parts, x, Wo[l], Wf)
        else:
            x, xf = _node_post(parts, x, Wo[l], Wf)

    gef, = _sc_gather([(xf, src)])
    msga, msgb = _edge_final(gef, rel_pos, scale2d, Rf1, Rf2)
    fpa = _sc_scatter_add(msga, dst, C, zeros64)
    fpb = _sc_scatter_add(msgb, dst, C, zeros64)
    return _final_sum(fpa, fpb)
